# baseline (speedup 1.0000x reference)
import functools

import jax
import jax.numpy as jnp
from jax import lax
from jax.experimental import pallas as pl
from jax.experimental.pallas import tpu as pltpu
from jax.experimental.pallas import tpu_sc as plsc

N_SPEAKERS = 100000
EMBED_DIM = 64
BATCH = 16384

_NC = 2
_NS = 16
_NW = _NC * _NS
_BPW = BATCH // _NW
_CH = 128
_NCH = _BPW // _CH

_mesh = plsc.VectorSubcoreMesh(core_axis_name="c", subcore_axis_name="s")


@functools.partial(
    pl.kernel,
    mesh=_mesh,
    out_type=jax.ShapeDtypeStruct((BATCH, EMBED_DIM), jnp.float32),
    scratch_types=[
        pltpu.VMEM((_BPW,), jnp.int32),
        pltpu.VMEM((_BPW, EMBED_DIM), jnp.float32),
    ] + [pltpu.SemaphoreType.DMA] * _NCH,
    compiler_params=pltpu.CompilerParams(use_tc_tiling_on_sc=False),
)
def _lookup(speaker_hbm, table_hbm, out_hbm, idx_v, rows_v, *sems):
    wid = lax.axis_index("s") * _NC + lax.axis_index("c")
    base = wid * _BPW
    pltpu.sync_copy(speaker_hbm.at[pl.ds(base, _BPW)], idx_v)
    copies = []
    for k in range(_NCH):
        copies.append(
            pltpu.async_copy(
                table_hbm.at[idx_v.at[pl.ds(k * _CH, _CH)]],
                rows_v.at[pl.ds(k * _CH, _CH)],
                sems[k],
            )
        )
    for c in copies:
        c.wait()
    pltpu.sync_copy(rows_v, out_hbm.at[pl.ds(base, _BPW)])


def kernel(speaker, table):
    return _lookup(speaker.astype(jnp.int32), table)


# speedup vs baseline: 1.0049x; 1.0049x over previous
import functools

import jax
import jax.numpy as jnp
from jax import lax
from jax.experimental import pallas as pl
from jax.experimental.pallas import tpu as pltpu
from jax.experimental.pallas import tpu_sc as plsc

N_SPEAKERS = 100000
EMBED_DIM = 64
BATCH = 16384

_NC = 2
_NS = 16
_NW = _NC * _NS
_BPW = BATCH // _NW
_CH = 128
_NCH = _BPW // _CH

_mesh = plsc.VectorSubcoreMesh(core_axis_name="c", subcore_axis_name="s")


@functools.partial(
    pl.kernel,
    mesh=_mesh,
    out_type=jax.ShapeDtypeStruct((BATCH, EMBED_DIM), jnp.float32),
    scratch_types=[
        pltpu.VMEM((_BPW,), jnp.int32),
        pltpu.VMEM((_BPW, EMBED_DIM), jnp.float32),
    ] + [pltpu.SemaphoreType.DMA] * _NCH,
    compiler_params=pltpu.CompilerParams(
        use_tc_tiling_on_sc=False,
        disable_bounds_checks=True,
        disable_semaphore_checks=True,
        skip_device_barrier=True,
    ),
)
def _lookup(speaker_hbm, table_hbm, out_hbm, idx_v, rows_v, *sems):
    wid = lax.axis_index("s") * _NC + lax.axis_index("c")
    base = wid * _BPW
    pltpu.sync_copy(speaker_hbm.at[pl.ds(base, _BPW)], idx_v)
    copies = []
    for k in range(_NCH):
        copies.append(
            pltpu.async_copy(
                table_hbm.at[idx_v.at[pl.ds(k * _CH, _CH)]],
                rows_v.at[pl.ds(k * _CH, _CH)],
                sems[k],
            )
        )
    for c in copies:
        c.wait()
    pltpu.sync_copy(rows_v, out_hbm.at[pl.ds(base, _BPW)])


def kernel(speaker, table):
    return _lookup(speaker.astype(jnp.int32), table)


# speedup vs baseline: 1.1498x; 1.1442x over previous
import functools

import jax
import jax.numpy as jnp
from jax import lax
from jax.experimental import pallas as pl
from jax.experimental.pallas import tpu as pltpu
from jax.experimental.pallas import tpu_sc as plsc

N_SPEAKERS = 100000
EMBED_DIM = 64
PAD_DIM = 128
BATCH = 16384

_NC = 2
_NS = 16
_NW = _NC * _NS
_BPW = BATCH // _NW
_CH = 128
_NCH = _BPW // _CH

_mesh = plsc.VectorSubcoreMesh(core_axis_name="c", subcore_axis_name="s")


@functools.partial(
    pl.kernel,
    mesh=_mesh,
    out_type=jax.ShapeDtypeStruct((BATCH, PAD_DIM), jnp.float32),
    scratch_types=[
        pltpu.VMEM((_BPW,), jnp.int32),
        pltpu.VMEM((_BPW, PAD_DIM), jnp.float32),
    ] + [pltpu.SemaphoreType.DMA] * _NCH,
)
def _lookup(speaker_hbm, table_hbm, out_hbm, idx_v, rows_v, *sems):
    wid = lax.axis_index("s") * _NC + lax.axis_index("c")
    base = wid * _BPW
    pltpu.sync_copy(speaker_hbm.at[pl.ds(base, _BPW)], idx_v)
    copies = []
    for k in range(_NCH):
        copies.append(
            pltpu.async_copy(
                table_hbm.at[idx_v.at[pl.ds(k * _CH, _CH)]],
                rows_v.at[pl.ds(k * _CH, _CH)],
                sems[k],
            )
        )
    for c in copies:
        c.wait()
    pltpu.sync_copy(rows_v, out_hbm.at[pl.ds(base, _BPW)])


def kernel(speaker, table):
    table_p = jnp.pad(table, ((0, 0), (0, PAD_DIM - EMBED_DIM)))
    out_p = _lookup(speaker.astype(jnp.int32), table_p)
    return out_p[:, :EMBED_DIM]
